# baseline (speedup 1.0000x reference)
import functools

import jax
import jax.numpy as jnp
import numpy as np
from jax import lax
from jax.experimental import pallas as pl
from jax.experimental.pallas import tpu as pltpu
from jax.experimental.pallas import tpu_sc as plsc

_B, _A, _N, _D, _NRBF, _ZMAX, _H = 8, 1024, 48, 128, 32, 100, 64
_E = _B * _A * _N
_NW = 32
_EPT = _E // _NW
_ROWS = _EPT // 128
_CPB = _NW // _B

_TA = 128
_NT = _A // _TA
_P = _TA * _N
_ST = _B * _NT


def _sc_body(idx_hbm, ctr_hbm, z_hbm, px_hbm, py_hbm, pz_hbm, zj_out, d2_out,
             idx_v, ctr_v, zj_v, d2_v, z_v, px_v, py_v, pz_v):
    c = lax.axis_index("c")
    s = lax.axis_index("s")
    w = s * 2 + c
    b = w // _CPB

    pltpu.sync_copy(idx_hbm.at[w], idx_v)
    pltpu.sync_copy(ctr_hbm.at[w], ctr_v)
    pltpu.sync_copy(z_hbm.at[b], z_v)
    pltpu.sync_copy(px_hbm.at[b], px_v)
    pltpu.sync_copy(py_hbm.at[b], py_v)
    pltpu.sync_copy(pz_hbm.at[b], pz_v)

    def body(row, carry):
        for k in range(8):
            col = k * 16
            iv = idx_v[row, pl.ds(col, 16)]
            zj_v[row, pl.ds(col, 16)] = plsc.load_gather(z_v, [iv])
            xg = plsc.load_gather(px_v, [iv])
            yg = plsc.load_gather(py_v, [iv])
            zg = plsc.load_gather(pz_v, [iv])
            aidx = ctr_v[row, pl.ds(col, 16)]
            cx = plsc.load_gather(px_v, [aidx])
            cy = plsc.load_gather(py_v, [aidx])
            cz = plsc.load_gather(pz_v, [aidx])
            dx = xg - cx
            dy = yg - cy
            dz = zg - cz
            d2_v[row, pl.ds(col, 16)] = dx * dx + dy * dy + dz * dz
        return carry

    lax.fori_loop(0, _ROWS, body, 0)

    pltpu.sync_copy(zj_v, zj_out.at[w])
    pltpu.sync_copy(d2_v, d2_out.at[w])


def _sc_edges(idx, ctr, z, px, py, pz):
    mesh = plsc.VectorSubcoreMesh(core_axis_name="c", subcore_axis_name="s")
    f = functools.partial(
        pl.kernel, mesh=mesh,
        compiler_params=pltpu.CompilerParams(needs_layout_passes=False),
        out_type=[
            jax.ShapeDtypeStruct((_NW, _ROWS, 128), jnp.int32),
            jax.ShapeDtypeStruct((_NW, _ROWS, 128), jnp.float32),
        ],
        scratch_types=[
            pltpu.VMEM((_ROWS, 128), jnp.int32),
            pltpu.VMEM((_ROWS, 128), jnp.int32),
            pltpu.VMEM((_ROWS, 128), jnp.int32),
            pltpu.VMEM((_ROWS, 128), jnp.float32),
            pltpu.VMEM((_A,), jnp.int32),
            pltpu.VMEM((_A,), jnp.float32),
            pltpu.VMEM((_A,), jnp.float32),
            pltpu.VMEM((_A,), jnp.float32),
        ],
    )(_sc_body)
    return f(idx, ctr, z, px, py, pz)


def _tc_body(zj_ref, d2_ref, z_ref, embT_ref, wf1T_ref, ohc_ref, wmsgT_ref,
             bmsg_ref, wo1T_ref, bo1_ref, wo2T_ref, out_ref,
             xT, embTbf, msgT_s):
    t = pl.program_id(1)

    @pl.when(t == 0)
    def _init():
        z_row = z_ref[0, 0][None, :]
        ohzc = (z_row == lax.broadcasted_iota(
            jnp.int32, (_ZMAX, _A), 0)).astype(jnp.float32)
        xT[...] = jnp.dot(embT_ref[...], ohzc,
                          preferred_element_type=jnp.float32)
        embTbf[...] = embT_ref[...].astype(jnp.bfloat16)

    zrow = zj_ref[0]
    ohz = (zrow == lax.broadcasted_iota(
        jnp.int32, (_ZMAX, _P), 0)).astype(jnp.bfloat16)
    xjT = jnp.dot(embTbf[...], ohz,
                  preferred_element_type=jnp.float32)

    d = jnp.sqrt(d2_ref[0] + 1e-8)
    centers = lax.broadcasted_iota(
        jnp.int32, (_NRBF, 1), 0).astype(jnp.float32) * (5.0 / (_NRBF - 1))
    delta = d - centers
    rbfT = jnp.exp(-10.0 * delta * delta)
    wijT = jnp.dot(wf1T_ref[...], rbfT,
                   preferred_element_type=jnp.float32)

    prodT = (xjT * wijT).astype(jnp.bfloat16)
    msgT_s[:, pl.ds(t * _TA, _TA)] = jnp.dot(
        prodT, ohc_ref[...], preferred_element_type=jnp.float32)

    @pl.when(t == _NT - 1)
    def _head():
        repT = xT[...] + jnp.tanh(
            jnp.dot(wmsgT_ref[...], msgT_s[...],
                    preferred_element_type=jnp.float32) + bmsg_ref[...])
        h1T = jnp.tanh(jnp.dot(wo1T_ref[...], repT,
                               preferred_element_type=jnp.float32)
                       + bo1_ref[...])
        atom_eT = jnp.dot(wo2T_ref[...], h1T,
                          preferred_element_type=jnp.float32)
        out_ref[...] = jnp.full((1, 1, 128), jnp.sum(atom_eT), jnp.float32)


def kernel(positions, atomic_numbers, neighbors, emb, W_f1, W_msg, b_msg,
           W_o1, b_o1, W_o2, b_o2):
    z = atomic_numbers.astype(jnp.int32)
    idx = neighbors.astype(jnp.int32).reshape(_NW, _ROWS, 128)
    px = positions[:, :, 0]
    py = positions[:, :, 1]
    pz = positions[:, :, 2]

    ctr = jnp.asarray(
        (np.arange(_E, dtype=np.int32) // _N) % _A).reshape(_NW, _ROWS, 128)
    zj, d2 = _sc_edges(idx, ctr, z, px, py, pz)
    zj = zj.reshape(_ST, 1, _P)
    d2 = d2.reshape(_ST, 1, _P)

    ohc = jnp.asarray(np.kron(np.eye(_TA, dtype=np.float32),
                              np.ones((1, _N), np.float32)).T
                      ).astype(jnp.bfloat16)

    grid = (_B, _NT)
    out = pl.pallas_call(
        _tc_body,
        grid=grid,
        in_specs=[
            pl.BlockSpec((1, 1, _P), lambda b, t: (b * _NT + t, 0, 0)),
            pl.BlockSpec((1, 1, _P), lambda b, t: (b * _NT + t, 0, 0)),
            pl.BlockSpec((1, 1, _A), lambda b, t: (b, 0, 0)),
            pl.BlockSpec((_D, _ZMAX), lambda b, t: (0, 0)),
            pl.BlockSpec((_D, _NRBF), lambda b, t: (0, 0)),
            pl.BlockSpec((_P, _TA), lambda b, t: (0, 0)),
            pl.BlockSpec((_D, _D), lambda b, t: (0, 0)),
            pl.BlockSpec((_D, 1), lambda b, t: (0, 0)),
            pl.BlockSpec((_H, _D), lambda b, t: (0, 0)),
            pl.BlockSpec((_H, 1), lambda b, t: (0, 0)),
            pl.BlockSpec((1, _H), lambda b, t: (0, 0)),
        ],
        out_specs=pl.BlockSpec((1, 1, 128), lambda b, t: (b, 0, 0)),
        out_shape=jax.ShapeDtypeStruct((_B, 1, 128), jnp.float32),
        scratch_shapes=[
            pltpu.VMEM((_D, _A), jnp.float32),
            pltpu.VMEM((_D, _ZMAX), jnp.bfloat16),
            pltpu.VMEM((_D, _A), jnp.float32),
        ],
    )(zj, d2, z.reshape(_B, 1, _A), emb.T, W_f1.T, ohc,
      W_msg.T, b_msg.reshape(_D, 1),
      W_o1.T, b_o1.reshape(_H, 1), W_o2.T)
    return out[:, 0, :1] + _A * b_o2[0]


# speedup vs baseline: 1.0441x; 1.0441x over previous
import functools

import jax
import jax.numpy as jnp
import numpy as np
from jax import lax
from jax.experimental import pallas as pl
from jax.experimental.pallas import tpu as pltpu
from jax.experimental.pallas import tpu_sc as plsc

_B, _A, _N, _D, _NRBF, _ZMAX, _H = 8, 1024, 48, 128, 32, 100, 64
_BH = 4
_EH = _BH * _A * _N
_NW = 32
_EPT = _EH // _NW
_ROWS = _EPT // 128
_CPB = _NW // _BH

_TA = 128
_NT = _A // _TA
_P = _TA * _N
_ST = _BH * _NT


def _sc_body(idx_hbm, ctr_hbm, z_hbm, px_hbm, py_hbm, pz_hbm, zj_out, d2_out,
             idx_v, ctr_v, zj_v, d2_v, z_v, px_v, py_v, pz_v):
    c = lax.axis_index("c")
    s = lax.axis_index("s")
    w = s * 2 + c
    b = w // _CPB

    pltpu.sync_copy(idx_hbm.at[w], idx_v)
    pltpu.sync_copy(ctr_hbm.at[w], ctr_v)
    pltpu.sync_copy(z_hbm.at[b], z_v)
    pltpu.sync_copy(px_hbm.at[b], px_v)
    pltpu.sync_copy(py_hbm.at[b], py_v)
    pltpu.sync_copy(pz_hbm.at[b], pz_v)

    def body(row, carry):
        for k in range(8):
            col = k * 16
            iv = idx_v[row, pl.ds(col, 16)]
            zj_v[row, pl.ds(col, 16)] = plsc.load_gather(z_v, [iv])
            xg = plsc.load_gather(px_v, [iv])
            yg = plsc.load_gather(py_v, [iv])
            zg = plsc.load_gather(pz_v, [iv])
            aidx = ctr_v[row, pl.ds(col, 16)]
            cx = plsc.load_gather(px_v, [aidx])
            cy = plsc.load_gather(py_v, [aidx])
            cz = plsc.load_gather(pz_v, [aidx])
            dx = xg - cx
            dy = yg - cy
            dz = zg - cz
            d2_v[row, pl.ds(col, 16)] = dx * dx + dy * dy + dz * dz
        return carry

    lax.fori_loop(0, _ROWS, body, 0)

    pltpu.sync_copy(zj_v, zj_out.at[w])
    pltpu.sync_copy(d2_v, d2_out.at[w])


def _sc_edges(idx, ctr, z, px, py, pz):
    mesh = plsc.VectorSubcoreMesh(core_axis_name="c", subcore_axis_name="s")
    f = functools.partial(
        pl.kernel, mesh=mesh,
        compiler_params=pltpu.CompilerParams(needs_layout_passes=False),
        out_type=[
            jax.ShapeDtypeStruct((_NW, _ROWS, 128), jnp.int32),
            jax.ShapeDtypeStruct((_NW, _ROWS, 128), jnp.float32),
        ],
        scratch_types=[
            pltpu.VMEM((_ROWS, 128), jnp.int32),
            pltpu.VMEM((_ROWS, 128), jnp.int32),
            pltpu.VMEM((_ROWS, 128), jnp.int32),
            pltpu.VMEM((_ROWS, 128), jnp.float32),
            pltpu.VMEM((_A,), jnp.int32),
            pltpu.VMEM((_A,), jnp.float32),
            pltpu.VMEM((_A,), jnp.float32),
            pltpu.VMEM((_A,), jnp.float32),
        ],
    )(_sc_body)
    return f(idx, ctr, z, px, py, pz)


def _tc_body(zj_ref, d2_ref, z_ref, embT_ref, wf1T_ref, ohc_ref, wmsgT_ref,
             bmsg_ref, wo1T_ref, bo1_ref, wo2T_ref, out_ref,
             xT, embTbf, msgT_s):
    t = pl.program_id(1)

    @pl.when(t == 0)
    def _init():
        z_row = z_ref[0, 0][None, :]
        ohzc = (z_row == lax.broadcasted_iota(
            jnp.int32, (_ZMAX, _A), 0)).astype(jnp.float32)
        xT[...] = jnp.dot(embT_ref[...], ohzc,
                          preferred_element_type=jnp.float32)
        embTbf[...] = embT_ref[...].astype(jnp.bfloat16)

    zrow = zj_ref[0]
    ohz = (zrow == lax.broadcasted_iota(
        jnp.int32, (_ZMAX, _P), 0)).astype(jnp.bfloat16)
    xjT = jnp.dot(embTbf[...], ohz,
                  preferred_element_type=jnp.float32)

    d = jnp.sqrt(d2_ref[0] + 1e-8)
    centers = lax.broadcasted_iota(
        jnp.int32, (_NRBF, 1), 0).astype(jnp.float32) * (5.0 / (_NRBF - 1))
    delta = d - centers
    rbfT = jnp.exp(-10.0 * delta * delta)
    wijT = jnp.dot(wf1T_ref[...], rbfT,
                   preferred_element_type=jnp.float32)

    prodT = (xjT * wijT).astype(jnp.bfloat16)
    msgT_s[:, pl.ds(t * _TA, _TA)] = jnp.dot(
        prodT, ohc_ref[...], preferred_element_type=jnp.float32)

    @pl.when(t == _NT - 1)
    def _head():
        repT = xT[...] + jnp.tanh(
            jnp.dot(wmsgT_ref[...], msgT_s[...],
                    preferred_element_type=jnp.float32) + bmsg_ref[...])
        h1T = jnp.tanh(jnp.dot(wo1T_ref[...], repT,
                               preferred_element_type=jnp.float32)
                       + bo1_ref[...])
        atom_eT = jnp.dot(wo2T_ref[...], h1T,
                          preferred_element_type=jnp.float32)
        out_ref[...] = jnp.full((1, 1, 128), jnp.sum(atom_eT), jnp.float32)


def _tc_half(zj, d2, z3, embT, wf1T, ohc, wmsgT, bmsgC, wo1T, bo1C, wo2T):
    grid = (_BH, _NT)
    return pl.pallas_call(
        _tc_body,
        grid=grid,
        in_specs=[
            pl.BlockSpec((1, 1, _P), lambda b, t: (b * _NT + t, 0, 0)),
            pl.BlockSpec((1, 1, _P), lambda b, t: (b * _NT + t, 0, 0)),
            pl.BlockSpec((1, 1, _A), lambda b, t: (b, 0, 0)),
            pl.BlockSpec((_D, _ZMAX), lambda b, t: (0, 0)),
            pl.BlockSpec((_D, _NRBF), lambda b, t: (0, 0)),
            pl.BlockSpec((_P, _TA), lambda b, t: (0, 0)),
            pl.BlockSpec((_D, _D), lambda b, t: (0, 0)),
            pl.BlockSpec((_D, 1), lambda b, t: (0, 0)),
            pl.BlockSpec((_H, _D), lambda b, t: (0, 0)),
            pl.BlockSpec((_H, 1), lambda b, t: (0, 0)),
            pl.BlockSpec((1, _H), lambda b, t: (0, 0)),
        ],
        out_specs=pl.BlockSpec((1, 1, 128), lambda b, t: (b, 0, 0)),
        out_shape=jax.ShapeDtypeStruct((_BH, 1, 128), jnp.float32),
        scratch_shapes=[
            pltpu.VMEM((_D, _A), jnp.float32),
            pltpu.VMEM((_D, _ZMAX), jnp.bfloat16),
            pltpu.VMEM((_D, _A), jnp.float32),
        ],
    )(zj, d2, z3, embT, wf1T, ohc, wmsgT, bmsgC, wo1T, bo1C, wo2T)


def kernel(positions, atomic_numbers, neighbors, emb, W_f1, W_msg, b_msg,
           W_o1, b_o1, W_o2, b_o2):
    z = atomic_numbers.astype(jnp.int32)
    nbr = neighbors.astype(jnp.int32)
    px = positions[:, :, 0]
    py = positions[:, :, 1]
    pz = positions[:, :, 2]

    ctr = jnp.asarray(
        (np.arange(_EH, dtype=np.int32) // _N) % _A).reshape(_NW, _ROWS, 128)
    ohc = jnp.asarray(np.kron(np.eye(_TA, dtype=np.float32),
                              np.ones((1, _N), np.float32)).T
                      ).astype(jnp.bfloat16)
    embT = emb.T
    wf1T = W_f1.T
    wmsgT = W_msg.T
    bmsgC = b_msg.reshape(_D, 1)
    wo1T = W_o1.T
    bo1C = b_o1.reshape(_H, 1)
    wo2T = W_o2.T

    outs = []
    for h in range(_B // _BH):
        sl = slice(h * _BH, (h + 1) * _BH)
        idx_h = nbr[sl].reshape(_NW, _ROWS, 128)
        zj, d2 = _sc_edges(idx_h, ctr, z[sl], px[sl], py[sl], pz[sl])
        outs.append(_tc_half(
            zj.reshape(_ST, 1, _P), d2.reshape(_ST, 1, _P),
            z[sl].reshape(_BH, 1, _A), embT, wf1T, ohc, wmsgT, bmsgC,
            wo1T, bo1C, wo2T))
    out = jnp.concatenate(outs, axis=0)
    return out[:, 0, :1] + _A * b_o2[0]
